# baseline (speedup 1.0000x reference)
import functools

import jax
import jax.numpy as jnp
from jax import lax
from jax.experimental import pallas as pl
from jax.experimental.pallas import tpu as pltpu
from jax.experimental.pallas import tpu_sc as plsc

_N_NODES = 10000
_N_EDGES = 20000
_N_INC = 320000
_D = 128
_DH = 64
_ALPHA = 0.5
_NTILES = 16
_PPT = _N_INC // _NTILES
_CH = 80
_NCH = _PPT // _CH
_NB = 50
_NBLK = _NCH // _NB
_ZCH = 160
_XE_PAD = 20480
_XE_PT = _XE_PAD // _NTILES
_S_PAD = 10240
_S_PT = _S_PAD // _NTILES
_DEGW = 8

_MESH = plsc.VectorSubcoreMesh(core_axis_name="c", subcore_axis_name="s")
_SC_PARAMS = pltpu.CompilerParams(use_tc_tiling_on_sc=False)


def _mm1_body(x_ref, w_ref, b_ref, o0_ref, o1_ref):
    y = jnp.dot(x_ref[...], w_ref[...], preferred_element_type=jnp.float32)
    y = y + b_ref[...]
    o0_ref[...] = y[:, :_DH]
    o1_ref[...] = y[:, _DH:]


def _xw1(X, W1_w, W1_b):
    blk = 1000
    return pl.pallas_call(
        _mm1_body,
        grid=(_N_NODES // blk,),
        in_specs=[
            pl.BlockSpec((blk, _D), lambda i: (i, 0)),
            pl.BlockSpec((_D, _D), lambda i: (0, 0)),
            pl.BlockSpec((1, _D), lambda i: (0, 0)),
        ],
        out_specs=[
            pl.BlockSpec((blk, _DH), lambda i: (i, 0)),
            pl.BlockSpec((blk, _DH), lambda i: (i, 0)),
        ],
        out_shape=[
            jax.ShapeDtypeStruct((_N_NODES, _DH), jnp.float32),
            jax.ShapeDtypeStruct((_N_NODES, _DH), jnp.float32),
        ],
    )(X, W1_w, W1_b)


def _sc1_body(xw1a, xw1b, vtx3, edg3, zrow, zdeg, ones_h,
              xe0, xe1, deg,
              vtx_v, edg_v, rowbuf, rowbuf2, ones_v, zbuf, zdbuf,
              xe_acc, deg_acc, gsem, gsem2):
    c = lax.axis_index("c")
    s = lax.axis_index("s")
    pltpu.sync_copy(zrow, zbuf)
    for k in range(_XE_PT // _ZCH):
        pltpu.sync_copy(zbuf, xe_acc.at[pl.ds(s * _XE_PT + k * _ZCH, _ZCH)])

    @pl.when(c == 0)
    def _():
        pltpu.sync_copy(zdeg, zdbuf)
        pltpu.sync_copy(zdbuf, deg_acc.at[pl.ds(s * _S_PT, _S_PT)])
        pltpu.sync_copy(ones_h, ones_v)

    plsc.subcore_barrier()

    def run(xw1h, with_deg):
        def block(b, carry):
            pltpu.sync_copy(vtx3.at[s, pl.ds(b * _NB, _NB)], vtx_v)
            pltpu.sync_copy(edg3.at[s, pl.ds(b * _NB, _NB)], edg_v)

            def pair(p, c2):
                j0 = 2 * p
                j1 = j0 + 1
                cp0 = pltpu.async_copy(xw1h.at[vtx_v.at[j0]], rowbuf, gsem)
                cp1 = pltpu.async_copy(xw1h.at[vtx_v.at[j1]], rowbuf2, gsem2)
                cp0.wait()
                pltpu.sync_copy(rowbuf, xe_acc.at[edg_v.at[j0]], add=True)
                if with_deg:
                    pltpu.sync_copy(ones_v, deg_acc.at[vtx_v.at[j0]], add=True)
                cp1.wait()
                pltpu.sync_copy(rowbuf2, xe_acc.at[edg_v.at[j1]], add=True)
                if with_deg:
                    pltpu.sync_copy(ones_v, deg_acc.at[vtx_v.at[j1]], add=True)
                return c2
            lax.fori_loop(0, _NB // 2, pair, 0)
            return carry
        lax.fori_loop(0, _NBLK, block, 0)

    @pl.when(c == 0)
    def _():
        run(xw1a, True)

    @pl.when(c == 1)
    def _():
        run(xw1b, False)

    plsc.subcore_barrier()

    def dump(xe_h):
        for k in range(_XE_PT // _ZCH):
            rows = pl.ds(s * _XE_PT + k * _ZCH, _ZCH)
            pltpu.sync_copy(xe_acc.at[rows], zbuf)
            pltpu.sync_copy(zbuf, xe_h.at[rows])

    @pl.when(c == 0)
    def _():
        dump(xe0)
        pltpu.sync_copy(deg_acc.at[pl.ds(s * _S_PT, _S_PT)], zdbuf)
        pltpu.sync_copy(zdbuf, deg.at[pl.ds(s * _S_PT, _S_PT)])

    @pl.when(c == 1)
    def _():
        dump(xe1)


_sc_phase1 = functools.partial(
    pl.kernel,
    out_type=[
        jax.ShapeDtypeStruct((_XE_PAD, _DH), jnp.float32),
        jax.ShapeDtypeStruct((_XE_PAD, _DH), jnp.float32),
        jax.ShapeDtypeStruct((_S_PAD, _DEGW), jnp.float32),
    ],
    mesh=_MESH,
    compiler_params=_SC_PARAMS,
    scratch_types=[
        pltpu.VMEM((_NB, _CH), jnp.int32),
        pltpu.VMEM((_NB, _CH), jnp.int32),
        pltpu.VMEM((_CH, _DH), jnp.float32),
        pltpu.VMEM((_CH, _DH), jnp.float32),
        pltpu.VMEM((_CH, _DEGW), jnp.float32),
        pltpu.VMEM((_ZCH, _DH), jnp.float32),
        pltpu.VMEM((_S_PT, _DEGW), jnp.float32),
        pltpu.VMEM_SHARED((_XE_PAD, _DH), jnp.float32),
        pltpu.VMEM_SHARED((_S_PAD, _DEGW), jnp.float32),
        pltpu.SemaphoreType.DMA,
        pltpu.SemaphoreType.DMA,
    ],
)(_sc1_body)


def _sc2_body(xe0, xe1, vtx3, edg3, zrow,
              s_out0, s_out1,
              vtx_v, edg_v, rowbuf, rowbuf2, zbuf, s_acc, gsem, gsem2):
    c = lax.axis_index("c")
    s = lax.axis_index("s")
    pltpu.sync_copy(zrow, zbuf)
    for k in range(_S_PT // _ZCH):
        pltpu.sync_copy(zbuf, s_acc.at[pl.ds(s * _S_PT + k * _ZCH, _ZCH)])
    plsc.subcore_barrier()

    def run(xe_h):
        def block(b, carry):
            pltpu.sync_copy(vtx3.at[s, pl.ds(b * _NB, _NB)], vtx_v)
            pltpu.sync_copy(edg3.at[s, pl.ds(b * _NB, _NB)], edg_v)

            def pair(p, c2):
                j0 = 2 * p
                j1 = j0 + 1
                cp0 = pltpu.async_copy(xe_h.at[edg_v.at[j0]], rowbuf, gsem)
                cp1 = pltpu.async_copy(xe_h.at[edg_v.at[j1]], rowbuf2, gsem2)
                cp0.wait()
                pltpu.sync_copy(rowbuf, s_acc.at[vtx_v.at[j0]], add=True)
                cp1.wait()
                pltpu.sync_copy(rowbuf2, s_acc.at[vtx_v.at[j1]], add=True)
                return c2
            lax.fori_loop(0, _NB // 2, pair, 0)
            return carry
        lax.fori_loop(0, _NBLK, block, 0)

    @pl.when(c == 0)
    def _():
        run(xe0)

    @pl.when(c == 1)
    def _():
        run(xe1)

    plsc.subcore_barrier()
    for k in range(_S_PT // _ZCH):
        rows = pl.ds(s * _S_PT + k * _ZCH, _ZCH)
        pltpu.sync_copy(s_acc.at[rows], zbuf)

        @pl.when(c == 0)
        def _():
            pltpu.sync_copy(zbuf, s_out0.at[rows])

        @pl.when(c == 1)
        def _():
            pltpu.sync_copy(zbuf, s_out1.at[rows])


_sc_phase2 = functools.partial(
    pl.kernel,
    out_type=[
        jax.ShapeDtypeStruct((_S_PAD, _DH), jnp.float32),
        jax.ShapeDtypeStruct((_S_PAD, _DH), jnp.float32),
    ],
    mesh=_MESH,
    compiler_params=_SC_PARAMS,
    scratch_types=[
        pltpu.VMEM((_NB, _CH), jnp.int32),
        pltpu.VMEM((_NB, _CH), jnp.int32),
        pltpu.VMEM((_CH, _DH), jnp.float32),
        pltpu.VMEM((_CH, _DH), jnp.float32),
        pltpu.VMEM((_ZCH, _DH), jnp.float32),
        pltpu.VMEM_SHARED((_S_PAD, _DH), jnp.float32),
        pltpu.SemaphoreType.DMA,
        pltpu.SemaphoreType.DMA,
    ],
)(_sc2_body)


def _final_body(x_ref, x0_ref, s0_ref, s1_ref, deg_ref,
                w2a_ref, w2b_ref, b2_ref, ww_ref, wb_ref, o_ref):
    d = deg_ref[...][:, 0:1]
    s_all = jnp.concatenate([s0_ref[...], s1_ref[...]], axis=-1)
    xv = jnp.dot(x_ref[...] * d, w2a_ref[...], preferred_element_type=jnp.float32)
    xv = xv + jnp.dot(s_all, w2b_ref[...], preferred_element_type=jnp.float32)
    xv = xv + d * b2_ref[...]
    xmix = (1.0 - _ALPHA) * xv + _ALPHA * x0_ref[...]
    o_ref[...] = jnp.dot(xmix, ww_ref[...], preferred_element_type=jnp.float32) + wb_ref[...]


def _final(X, X0, S0, S1, deg, W2a, W2b, b2, W_w, W_b):
    blk = 1000
    full = lambda i: (0, 0)
    return pl.pallas_call(
        _final_body,
        grid=(_N_NODES // blk,),
        in_specs=[
            pl.BlockSpec((blk, _D), lambda i: (i, 0)),
            pl.BlockSpec((blk, _D), lambda i: (i, 0)),
            pl.BlockSpec((blk, _DH), lambda i: (i, 0)),
            pl.BlockSpec((blk, _DH), lambda i: (i, 0)),
            pl.BlockSpec((blk, _DEGW), lambda i: (i, 0)),
            pl.BlockSpec((_D, _D), full),
            pl.BlockSpec((_D, _D), full),
            pl.BlockSpec((1, _D), full),
            pl.BlockSpec((_D, _D), full),
            pl.BlockSpec((1, _D), full),
        ],
        out_specs=pl.BlockSpec((blk, _D), lambda i: (i, 0)),
        out_shape=jax.ShapeDtypeStruct((_N_NODES, _D), jnp.float32),
    )(X, X0, S0, S1, deg, W2a, W2b, b2, W_w, W_b)


def kernel(X, vertex, edges, X0, W1_w, W1_b, W2_w, W2_b, W_w, W_b):
    vertex = vertex.astype(jnp.int32)
    edges = edges.astype(jnp.int32)
    vtx3 = vertex.reshape(_NTILES, _NCH, _CH)
    edg3 = edges.reshape(_NTILES, _NCH, _CH)

    xw1a, xw1b = _xw1(X, W1_w, W1_b.reshape(1, _D))

    zrow = jnp.zeros((_ZCH, _DH), jnp.float32)
    zdeg = jnp.zeros((_S_PT, _DEGW), jnp.float32)
    ones_h = jnp.ones((_CH, _DEGW), jnp.float32)

    xe0, xe1, deg = _sc_phase1(xw1a, xw1b, vtx3, edg3, zrow, zdeg, ones_h)
    S0, S1 = _sc_phase2(xe0, xe1, vtx3, edg3, zrow)
    S0 = S0[:_N_NODES]
    S1 = S1[:_N_NODES]
    deg = deg[:_N_NODES]

    out = _final(X, X0, S0, S1, deg,
                 W2_w[:_D], W2_w[_D:],
                 W2_b.reshape(1, _D), W_w, W_b.reshape(1, _D))
    return out
